# baseline (speedup 1.0000x reference)
import functools
import jax
import jax.numpy as jnp
from jax import lax
from jax.experimental import pallas as pl
from jax.experimental.pallas import tpu as pltpu
from jax.experimental.pallas import tpu_sc as plsc

_D = 64
_C = 1024
_B = 16
_T = 1024
_N = _B * _T
_U = 8


def _tc1_body(x_ref, e_ref, idx_ref, pplx_ref, counts_ref):
    b = pl.program_id(0)
    e = e_ref[...]
    e_bf = e.astype(jnp.bfloat16)
    em2_bf = e_bf * jnp.bfloat16(-2.0)
    e2 = jnp.sum(e * e, axis=0)
    e2_col = e2[None, :].T

    @pl.when(b == 0)
    def _init():
        counts_ref[...] = jnp.zeros_like(counts_ref)

    cnt = jnp.zeros((_C, 1), jnp.float32)
    for i in range(_U):
        xb = x_ref[i]
        xem2 = jax.lax.dot_general(em2_bf, xb.astype(jnp.bfloat16),
                                   (((0,), (0,)), ((), ())),
                                   preferred_element_type=jnp.float32)
        x2 = jnp.sum(xb * xb, axis=0)
        dist = (x2[None, :] + xem2) + e2_col
        idx = jnp.argmin(dist, axis=0).astype(jnp.int32)
        idx_ref[i] = idx
        oh_bf = (jax.lax.broadcasted_iota(jnp.int32, (_C, _T), 0)
                 == idx[None, :]).astype(jnp.bfloat16)
        ones_col = jnp.ones((_T, 1), jnp.bfloat16)
        cnt = cnt + jax.lax.dot_general(oh_bf, ones_col,
                                        (((1,), (0,)), ((), ())),
                                        preferred_element_type=jnp.float32)

    counts_ref[...] += cnt

    @pl.when(b == (_B // _U) - 1)
    def _fin():
        probs = counts_ref[...] * (1.0 / _N)
        ent = -jnp.sum(probs * jnp.log(probs + 1e-10))
        pplx_ref[...] = jnp.exp(ent).reshape(1, 1)


_tc1 = pl.pallas_call(
    _tc1_body,
    grid=(_B // _U,),
    in_specs=[
        pl.BlockSpec((_U, _D, _T), lambda b: (b, 0, 0)),
        pl.BlockSpec((_D, _C), lambda b: (0, 0)),
    ],
    out_specs=[
        pl.BlockSpec((_U, _T), lambda b: (b, 0)),
        pl.BlockSpec((1, 1), lambda b: (0, 0)),
    ],
    out_shape=[
        jax.ShapeDtypeStruct((_B, _T), jnp.int32),
        jax.ShapeDtypeStruct((1, 1), jnp.float32),
    ],
    scratch_shapes=[pltpu.VMEM((_C, 1), jnp.float32)],
)


def _sc_gather(table, idx_flat):
    info = plsc.get_sparse_core_info()
    nc, ns = info.num_cores, info.num_subcores
    nw = nc * ns
    bpw = _N // nw
    k_chunks = bpw // 128

    @functools.partial(
        pl.kernel,
        mesh=plsc.VectorSubcoreMesh(core_axis_name="c", subcore_axis_name="s"),
        out_type=jax.ShapeDtypeStruct((_N, 128), jnp.float32),
        scratch_types=[
            pltpu.VMEM((k_chunks, 128), jnp.int32),
            pltpu.VMEM((bpw, 128), jnp.float32),
            pltpu.SemaphoreType.DMA,
        ],
    )
    def k(table_hbm, idx_hbm, out_hbm, idx_v, rows_v, sem):
        wid = lax.axis_index("s") * nc + lax.axis_index("c")
        base = wid * bpw
        pltpu.sync_copy(idx_hbm.at[pl.ds(wid * k_chunks, k_chunks)], idx_v)
        for j in range(k_chunks):
            pltpu.async_copy(table_hbm.at[idx_v.at[j]],
                             rows_v.at[pl.ds(j * 128, 128)], sem).wait()
        pltpu.sync_copy(rows_v, out_hbm.at[pl.ds(base, bpw)])

    return k(table, idx_flat.reshape(_N // 128, 128))


def _tc2_body(x_ref, qf_ref, q_ref):
    xb = x_ref[0]
    qT = qf_ref[0][:, :_D].T
    q_ref[0] = xb + (qT - xb)


_tc2 = pl.pallas_call(
    _tc2_body,
    grid=(_B,),
    in_specs=[
        pl.BlockSpec((1, _D, _T), lambda b: (b, 0, 0)),
        pl.BlockSpec((1, _T, 128), lambda b: (b, 0, 0)),
    ],
    out_specs=pl.BlockSpec((1, _D, _T), lambda b: (b, 0, 0)),
    out_shape=jax.ShapeDtypeStruct((_B, _D, _T), jnp.float32),
)


def kernel(x, embed):
    idx, pplx = _tc1(x, embed)
    table = jnp.pad(jnp.swapaxes(embed, 0, 1), ((0, 0), (0, 128 - _D)))
    qflat = _sc_gather(table, idx.reshape(-1))
    q = _tc2(x, qflat.reshape(_B, _T, 128))
    return q, idx, pplx[0, 0]


# speedup vs baseline: 3.6627x; 3.6627x over previous
import jax
import jax.numpy as jnp
from jax.experimental import pallas as pl
from jax.experimental.pallas import tpu as pltpu

_D = 64
_C = 1024
_B = 16
_T = 1024
_N = _B * _T
_U = 8


def _vq_body(x_ref, e_ref, q_ref, idx_ref, pplx_ref, counts_ref):
    b = pl.program_id(0)
    e = e_ref[...]
    e_bf = e.astype(jnp.bfloat16)
    em2_bf = e_bf * jnp.bfloat16(-2.0)
    e2 = jnp.sum(e * e, axis=0)
    e2_col = e2[None, :].T

    @pl.when(b == 0)
    def _init():
        counts_ref[...] = jnp.zeros_like(counts_ref)

    cnt = jnp.zeros((_C, 1), jnp.float32)
    for i in range(_U):
        xb = x_ref[i]
        xem2 = jax.lax.dot_general(em2_bf, xb.astype(jnp.bfloat16),
                                   (((0,), (0,)), ((), ())),
                                   preferred_element_type=jnp.float32)
        x2 = jnp.sum(xb * xb, axis=0)
        dist = (x2[None, :] + xem2) + e2_col
        idx = jnp.argmin(dist, axis=0).astype(jnp.int32)
        idx_ref[i] = idx
        oh_bf = (jax.lax.broadcasted_iota(jnp.int32, (_C, _T), 0)
                 == idx[None, :]).astype(jnp.bfloat16)
        q = jax.lax.dot_general(e_bf, oh_bf,
                                (((1,), (0,)), ((), ())),
                                preferred_element_type=jnp.float32)
        q_ref[i] = xb + (q - xb)
        cnt = cnt + jnp.sum(oh_bf.astype(jnp.float32), axis=1, keepdims=True)

    counts_ref[...] += cnt

    @pl.when(b == (_B // _U) - 1)
    def _fin():
        probs = counts_ref[...] * (1.0 / _N)
        ent = -jnp.sum(probs * jnp.log(probs + 1e-10))
        pplx_ref[...] = jnp.exp(ent).reshape(1, 1)


def _make_call(interpret=False):
    return pl.pallas_call(
        _vq_body,
        grid=(_B // _U,),
        in_specs=[
            pl.BlockSpec((_U, _D, _T), lambda b: (b, 0, 0)),
            pl.BlockSpec((_D, _C), lambda b: (0, 0)),
        ],
        out_specs=[
            pl.BlockSpec((_U, _D, _T), lambda b: (b, 0, 0)),
            pl.BlockSpec((_U, _T), lambda b: (b, 0)),
            pl.BlockSpec((1, 1), lambda b: (0, 0)),
        ],
        out_shape=[
            jax.ShapeDtypeStruct((_B, _D, _T), jnp.float32),
            jax.ShapeDtypeStruct((_B, _T), jnp.int32),
            jax.ShapeDtypeStruct((1, 1), jnp.float32),
        ],
        scratch_shapes=[pltpu.VMEM((_C, 1), jnp.float32)],
        interpret=interpret,
    )


def kernel(x, embed):
    q, idx, pplx = _make_call()(x, embed)
    return q, idx, pplx[0, 0]
